# baseline (speedup 1.0000x reference)
import functools

import jax
import jax.numpy as jnp
from jax import lax
from jax.experimental import pallas as pl
from jax.experimental.pallas import tpu as pltpu
from jax.experimental.pallas import tpu_sc as plsc

VOCAB = 1000000
DIM = 64
B = 4096 * 50
NW = 32
CHUNK = 128
PER_W = B // NW
NCHUNK = PER_W // CHUNK


def _gather_kernel(table_hbm, idx_hbm, out_hbm, idx_v, rows_v, sem):
    wid = lax.axis_index("s") * 2 + lax.axis_index("c")
    pltpu.sync_copy(idx_hbm.at[wid], idx_v)
    base = wid * PER_W

    def body(j, carry):
        pltpu.async_copy(table_hbm.at[idx_v.at[j]], rows_v, sem).wait()
        pltpu.sync_copy(rows_v, out_hbm.at[pl.ds(base + j * CHUNK, CHUNK)])
        return carry

    lax.fori_loop(0, NCHUNK, body, 0)


@jax.jit
def _embed(x_flat, table):
    mesh = plsc.VectorSubcoreMesh(core_axis_name="c", subcore_axis_name="s")
    k = functools.partial(
        pl.kernel,
        out_type=jax.ShapeDtypeStruct((B, DIM), jnp.float32),
        mesh=mesh,
        scratch_types=[
            pltpu.VMEM((NCHUNK, CHUNK), jnp.int32),
            pltpu.VMEM((CHUNK, DIM), jnp.float32),
            pltpu.SemaphoreType.DMA,
        ],
        compiler_params=pltpu.CompilerParams(use_tc_tiling_on_sc=False),
    )(_gather_kernel)
    return k(table, x_flat)


def kernel(x, table):
    x_flat = x.astype(jnp.int32).reshape(NW, NCHUNK, CHUNK)
    out = _embed(x_flat, table)
    return out.reshape(x.shape + (DIM,))


# speedup vs baseline: 1.0449x; 1.0449x over previous
import functools

import jax
import jax.numpy as jnp
from jax import lax
from jax.experimental import pallas as pl
from jax.experimental.pallas import tpu as pltpu
from jax.experimental.pallas import tpu_sc as plsc

VOCAB = 1000000
DIM = 64
B = 4096 * 50
NW = 32
CHUNK = 128
PER_W = B // NW
NCHUNK = PER_W // CHUNK
R = 10
F = 5
STEPS = NCHUNK // R


def _gather_kernel(table_hbm, idx_hbm, out_hbm, idx_v, bufs, gsem, wbsem):
    wid = lax.axis_index("s") * 2 + lax.axis_index("c")
    pltpu.sync_copy(idx_hbm.at[wid], idx_v)
    base = wid * PER_W

    def fire(t, b):
        pltpu.async_copy(table_hbm.at[idx_v.at[t]], bufs.at[b], gsem.at[b])

    def wait_gather(t, b):
        pltpu.make_async_copy(
            table_hbm.at[idx_v.at[t]], bufs.at[b], gsem.at[b]).wait()

    def start_wb(t, b):
        pltpu.async_copy(
            bufs.at[b], out_hbm.at[pl.ds(base + t * CHUNK, CHUNK)],
            wbsem.at[b])

    def wait_wb(t, b):
        pltpu.make_async_copy(
            bufs.at[b], out_hbm.at[pl.ds(base + t * CHUNK, CHUNK)],
            wbsem.at[b]).wait()

    for b in range(F):
        fire(b, b)

    for b in range(R):
        t = b
        wait_gather(t, b)
        start_wb(t, b)
        t2 = t + F
        b2 = (b + F) % R
        if t2 >= R:
            wait_wb(t2 - R, b2)
        fire(t2, b2)

    def step_body(s, carry):
        for b in range(R):
            t = s * R + b
            wait_gather(t, b)
            start_wb(t, b)
            t2 = t + F
            b2 = (b + F) % R
            wait_wb(t2 - R, b2)
            fire(t2, b2)
        return carry

    lax.fori_loop(1, STEPS - 1, step_body, 0)

    for b in range(R):
        t = (STEPS - 1) * R + b
        wait_gather(t, b)
        start_wb(t, b)
        t2 = t + F
        if t2 < NCHUNK:
            b2 = (b + F) % R
            wait_wb(t2 - R, b2)
            fire(t2, b2)

    for b in range(R):
        wait_wb((STEPS - 1) * R + b, b)


@jax.jit
def _embed(x_flat, table):
    mesh = plsc.VectorSubcoreMesh(core_axis_name="c", subcore_axis_name="s")
    k = functools.partial(
        pl.kernel,
        out_type=jax.ShapeDtypeStruct((B, DIM), jnp.float32),
        mesh=mesh,
        scratch_types=[
            pltpu.VMEM((NCHUNK, CHUNK), jnp.int32),
            pltpu.VMEM((R, CHUNK, DIM), jnp.float32),
            pltpu.SemaphoreType.DMA((R,)),
            pltpu.SemaphoreType.DMA((R,)),
        ],
        compiler_params=pltpu.CompilerParams(use_tc_tiling_on_sc=False),
    )(_gather_kernel)
    return k(table, x_flat)


def kernel(x, table):
    x_flat = x.astype(jnp.int32).reshape(NW, NCHUNK, CHUNK)
    out = _embed(x_flat, table)
    return out.reshape(x.shape + (DIM,))
